# Initial kernel scaffold; baseline (speedup 1.0000x reference)
import jax
import jax.numpy as jnp
from jax.experimental import pallas as pl


def kernel(x, edge_index, train_edge_id, fc1_W, fc1_b, gat1_W, gat1_as, gat1_ad, gat1_bias, bn1_g, bn1_b, fc5_W, fc5_b, gat2_W, gat2_as, gat2_ad, gat2_bias, bn2_g, bn2_b, fc2_W, fc2_b, fc4_W, fc4_b):
    raise NotImplementedError("write your pallas kernel here")


# speedup vs baseline: 7.6980x; 7.6980x over previous
import functools

import jax
import jax.numpy as jnp
import numpy as np
from jax import lax
from jax.experimental import pallas as pl
from jax.experimental.pallas import tpu as pltpu
from jax.experimental.pallas import tpu_sc as plsc

_F32 = jnp.float32
_I32 = jnp.int32

_B = 128
_NW = 32
_RB = 2000


def _tc1(x, w1, b1, gw, ps, pd, N):
    nb = N // _RB

    def body(x_ref, w1_ref, b1_ref, gw_ref, ps_ref, pd_ref,
             h_ref, as_ref, ad_ref):
        xg = jnp.dot(x_ref[...], w1_ref[...],
                     preferred_element_type=_F32, precision=jax.lax.Precision.HIGHEST) + b1_ref[...]
        h = jnp.dot(xg, gw_ref[...], preferred_element_type=_F32, precision=jax.lax.Precision.HIGHEST)
        h_ref[...] = h
        as_ref[...] = jnp.dot(h, ps_ref[...], preferred_element_type=_F32, precision=jax.lax.Precision.HIGHEST)
        ad_ref[...] = jnp.dot(h, pd_ref[...], preferred_element_type=_F32, precision=jax.lax.Precision.HIGHEST)

    D = x.shape[1]
    full = lambda a: pl.BlockSpec(a.shape, lambda i: (0,) * a.ndim)
    return pl.pallas_call(
        body,
        grid=(nb,),
        in_specs=[pl.BlockSpec((_RB, D), lambda i: (i, 0)),
                  full(w1), full(b1), full(gw), full(ps), full(pd)],
        out_specs=[pl.BlockSpec((_RB, D), lambda i: (i, 0)),
                   pl.BlockSpec((_RB, 16), lambda i: (i, 0)),
                   pl.BlockSpec((_RB, 16), lambda i: (i, 0))],
        out_shape=[jax.ShapeDtypeStruct((N, D), _F32),
                   jax.ShapeDtypeStruct((N, 16), _F32),
                   jax.ShapeDtypeStruct((N, 16), _F32)],
    )(x, w1, b1, gw, ps, pd)


def _tc_mid(raw, den, res, rep, gbias, bng, bnb, wn, bn_, gw2, ps2, pd2, N,
            last):
    nb = N // _RB
    Din = raw.shape[1]
    Dout = wn.shape[1]

    def body(raw_ref, den_ref, res_ref, rep_ref, gb_ref, g_ref, b_ref,
             wn_ref, bn_ref, gw2_ref, ps2_ref, pd2_ref, *o_refs):
        acc_s, acc_ss = o_refs[-2], o_refs[-1]
        outs = o_refs[:-2]
        p = pl.program_id(0)
        i = pl.program_id(1)
        dexp = jnp.dot(den_ref[...], rep_ref[...], preferred_element_type=_F32, precision=jax.lax.Precision.HIGHEST)
        y = raw_ref[...] / (dexp + 1e-16) + gb_ref[...]

        @pl.when(p == 0)
        def _():
            bs = jnp.sum(y, axis=0, keepdims=True)
            bss = jnp.sum(y * y, axis=0, keepdims=True)

            @pl.when(i == 0)
            def _():
                acc_s[...] = bs
                acc_ss[...] = bss

            @pl.when(i > 0)
            def _():
                acc_s[...] += bs
                acc_ss[...] += bss

        @pl.when(p == 1)
        def _():
            mu = acc_s[...] / N
            var = acc_ss[...] / N - mu * mu
            ybn = g_ref[...] * (y - mu) * lax.rsqrt(var + 1e-5) + b_ref[...]
            h = jnp.maximum(res_ref[...] + ybn, 0.0)
            h2 = jnp.dot(h, wn_ref[...], preferred_element_type=_F32, precision=jax.lax.Precision.HIGHEST) + bn_ref[...]
            outs[0][...] = h2
            if not last:
                hh = jnp.dot(h2, gw2_ref[...], preferred_element_type=_F32, precision=jax.lax.Precision.HIGHEST)
                outs[1][...] = hh
                outs[2][...] = jnp.dot(hh, ps2_ref[...],
                                       preferred_element_type=_F32, precision=jax.lax.Precision.HIGHEST)
                outs[3][...] = jnp.dot(hh, pd2_ref[...],
                                       preferred_element_type=_F32, precision=jax.lax.Precision.HIGHEST)

    full = lambda a: pl.BlockSpec(a.shape, lambda p, i: (0,) * a.ndim)
    out_shape = [jax.ShapeDtypeStruct((N, Dout), _F32)]
    out_specs = [pl.BlockSpec((_RB, Dout), lambda p, i: (i, 0))]
    if not last:
        out_shape += [jax.ShapeDtypeStruct((N, Dout), _F32),
                      jax.ShapeDtypeStruct((N, 16), _F32),
                      jax.ShapeDtypeStruct((N, 16), _F32)]
        out_specs += [pl.BlockSpec((_RB, Dout), lambda p, i: (i, 0)),
                      pl.BlockSpec((_RB, 16), lambda p, i: (i, 0)),
                      pl.BlockSpec((_RB, 16), lambda p, i: (i, 0))]
    return pl.pallas_call(
        body,
        grid=(2, nb),
        in_specs=[pl.BlockSpec((_RB, Din), lambda p, i: (i, 0)),
                  pl.BlockSpec((_RB, 16), lambda p, i: (i, 0)),
                  pl.BlockSpec((_RB, Din), lambda p, i: (i, 0)),
                  full(rep), full(gbias), full(bng), full(bnb),
                  full(wn), full(bn_), full(gw2), full(ps2), full(pd2)],
        out_specs=out_specs,
        out_shape=out_shape,
        scratch_shapes=[pltpu.VMEM((1, Din), _F32), pltpu.VMEM((1, Din), _F32)],
    )(raw, den, res, rep, gbias, bng, bnb, wn, bn_, gw2, ps2, pd2)


def _tc4(p, w, b, T):
    nb = T // _RB

    def body(x1_ref, x2_ref, w_ref, b_ref, o_ref):
        o_ref[...] = jnp.dot(x1_ref[...] * x2_ref[...], w_ref[...],
                             preferred_element_type=_F32, precision=jax.lax.Precision.HIGHEST) + b_ref[...]

    x1, x2 = p
    full = lambda a: pl.BlockSpec(a.shape, lambda i: (0,) * a.ndim)
    return pl.pallas_call(
        body,
        grid=(nb,),
        in_specs=[pl.BlockSpec((_RB, 256), lambda i: (i, 0)),
                  pl.BlockSpec((_RB, 256), lambda i: (i, 0)),
                  full(w), full(b)],
        out_specs=pl.BlockSpec((_RB, 128), lambda i: (i, 0)),
        out_shape=jax.ShapeDtypeStruct((T, 128), _F32),
    )(x1, x2, w, b)


def _gat_agg(h, as_p, ad_p, src2, dst2, zD, z16, CH, hd0, rng):
    D = h.shape[1]
    E2P = src2.shape[0]
    nch = E2P // (_B * 16)
    rps = rng // 16
    slab_rows = rng + 8
    heads = D // CH
    mesh = plsc.VectorSubcoreMesh(core_axis_name="c", subcore_axis_name="s")

    @functools.partial(
        pl.kernel,
        out_type=(jax.ShapeDtypeStruct((2 * rng, D), _F32),
                  jax.ShapeDtypeStruct((2 * rng, 16), _F32)),
        mesh=mesh,
        compiler_params=pltpu.CompilerParams(use_tc_tiling_on_sc=False),
        scratch_types=[
            pltpu.VMEM((_B,), _I32),
            pltpu.VMEM((_B,), _I32),
            pltpu.VMEM((_B,), _I32),
            pltpu.VMEM((_B, 16), _F32),
            pltpu.VMEM((_B, 16), _F32),
            pltpu.VMEM((_B, 16), _F32),
            pltpu.VMEM((_B, D), _F32),
            pltpu.VMEM_SHARED((slab_rows, D), _F32),
            pltpu.VMEM_SHARED((slab_rows, 16), _F32),
        ],
    )
    def k(h_hbm, as_hbm, ad_hbm, src_hbm, dst_hbm, zD_hbm, z16_hbm,
          raw_hbm, den_hbm,
          srcv, dstv, idxl, asr, adr, wbuf, hrows, slab, dslab):
        c = lax.axis_index("c")
        s = lax.axis_index("s")
        base = c * rng
        row0 = s * rps
        pltpu.sync_copy(zD_hbm, slab.at[pl.ds(row0, rps)])
        pltpu.sync_copy(z16_hbm, dslab.at[pl.ds(row0, rps)])
        plsc.subcore_barrier()

        @pl.loop(0, nch)
        def _(g):
            e0 = (g * 16 + s) * _B
            pltpu.sync_copy(src_hbm.at[pl.ds(e0, _B)], srcv)
            pltpu.sync_copy(dst_hbm.at[pl.ds(e0, _B)], dstv)
            pltpu.sync_copy(as_hbm.at[srcv], asr)
            pltpu.sync_copy(ad_hbm.at[dstv], adr)
            pltpu.sync_copy(h_hbm.at[srcv], hrows)

            @pl.loop(0, _B, step=16)
            def _(i):
                dl = dstv[pl.ds(i, 16)] - base
                inr = (dl >= 0) & (dl < rng)
                idxl[pl.ds(i, 16)] = jnp.where(inr, dl, rng)

            @pl.loop(0, _B)
            def _(e):
                l16 = asr[e, pl.ds(0, 16)] + adr[e, pl.ds(0, 16)]
                w16 = jnp.exp(jnp.maximum(l16, 0.2 * l16))
                wbuf[e, pl.ds(0, 16)] = w16
                for hd in range(heads):
                    wsc = w16[hd0 + hd]
                    for j in range(CH // 16):
                        sl = pl.ds(hd * CH + j * 16, 16)
                        hrows[e, sl] = hrows[e, sl] * wsc

            pltpu.sync_copy(hrows, slab.at[idxl], add=True)
            pltpu.sync_copy(wbuf, dslab.at[idxl], add=True)

        plsc.subcore_barrier()
        pltpu.sync_copy(slab.at[pl.ds(row0, rps)],
                        raw_hbm.at[pl.ds(base + row0, rps)])
        pltpu.sync_copy(dslab.at[pl.ds(row0, rps)],
                        den_hbm.at[pl.ds(base + row0, rps)])

    return k(h, as_p, ad_p, src2, dst2, zD, z16)


def _sc_pairs(hf, pairs, tid, T3):
    nch = T3 // (_B * _NW)
    mesh = plsc.VectorSubcoreMesh(core_axis_name="c", subcore_axis_name="s")

    @functools.partial(
        pl.kernel,
        out_type=(jax.ShapeDtypeStruct((T3, 256), _F32),
                  jax.ShapeDtypeStruct((T3, 256), _F32)),
        mesh=mesh,
        compiler_params=pltpu.CompilerParams(use_tc_tiling_on_sc=False,
                                             needs_layout_passes=False),
        scratch_types=[
            pltpu.VMEM((_B,), _I32),
            pltpu.VMEM((_B, 16), _I32),
            pltpu.VMEM((_B,), _I32),
            pltpu.VMEM((_B,), _I32),
            pltpu.VMEM((_B, 256), _F32),
            pltpu.VMEM((_B, 256), _F32),
        ],
    )
    def k(hf_hbm, pairs_hbm, tid_hbm, o1_hbm, o2_hbm,
          tidv, prow, srcv, dstv, x1, x2):
        c = lax.axis_index("c")
        s = lax.axis_index("s")
        wid = s * 2 + c

        @pl.loop(0, nch)
        def _(g):
            t0 = (g * _NW + wid) * _B
            pltpu.sync_copy(tid_hbm.at[pl.ds(t0, _B)], tidv)
            pltpu.sync_copy(pairs_hbm.at[tidv], prow)

            @pl.loop(0, _B, step=16)
            def _(i):
                ri = jnp.arange(16, dtype=_I32) + i
                srcv[pl.ds(i, 16)] = plsc.load_gather(
                    prow, [ri, jnp.zeros((16,), _I32)])
                dstv[pl.ds(i, 16)] = plsc.load_gather(
                    prow, [ri, jnp.ones((16,), _I32)])

            pltpu.sync_copy(hf_hbm.at[srcv], x1)
            pltpu.sync_copy(hf_hbm.at[dstv], x2)
            pltpu.sync_copy(x1, o1_hbm.at[pl.ds(t0, _B)])
            pltpu.sync_copy(x2, o2_hbm.at[pl.ds(t0, _B)])

    return k(hf, pairs, tid)


def _blockdiag(att, heads, ch):
    eye = jnp.eye(heads, dtype=_F32)
    P = (eye[:, None, :] * att[:, :, None]).reshape(heads * ch, heads)
    return jnp.pad(P, ((0, 0), (0, 16 - heads)))


def kernel(x, edge_index, train_edge_id, fc1_W, fc1_b, gat1_W, gat1_as,
           gat1_ad, gat1_bias, bn1_g, bn1_b, fc5_W, fc5_b, gat2_W, gat2_as,
           gat2_ad, gat2_bias, bn2_g, bn2_b, fc2_W, fc2_b, fc4_W, fc4_b):
    N = x.shape[0]
    E = edge_index.shape[1]
    T = train_edge_id.shape[0]
    rng = -(-(-(-N // 2)) // 128) * 128
    NPAD = 2 * rng
    pad_dst = N + 8
    E2 = E + N
    E2P = -(-E2 // (_B * _NW)) * (_B * _NW)
    T3 = -(-T // (_B * _NW)) * (_B * _NW)

    loop = jnp.arange(N, dtype=_I32)
    src2 = jnp.concatenate([edge_index[0], loop,
                            jnp.zeros((E2P - E2,), _I32)])
    dst2 = jnp.concatenate([edge_index[1], loop,
                            jnp.full((E2P - E2,), pad_dst, _I32)])
    pairs = jnp.zeros((E, 16), _I32)
    pairs = pairs.at[:, 0].set(edge_index[0]).at[:, 1].set(edge_index[1])
    tidp = jnp.concatenate([train_edge_id, jnp.zeros((T3 - T,), _I32)])

    rps = rng // 16
    z192 = jnp.zeros((rps, 192), _F32)
    z256 = jnp.zeros((rps, 256), _F32)
    z16 = jnp.zeros((rps, 16), _F32)

    ps1 = _blockdiag(gat1_as, 8, 48)
    pd1 = _blockdiag(gat1_ad, 8, 48)
    ps2 = _blockdiag(gat2_as, 1, 256)
    pd2 = _blockdiag(gat2_ad, 1, 256)
    rep1 = jnp.asarray(np.pad(np.kron(np.eye(8, dtype=np.float32),
                                      np.ones((1, 48), np.float32)),
                              ((0, 8), (0, 0))))
    rep2 = jnp.asarray(np.pad(np.ones((1, 256), np.float32), ((0, 15), (0, 0))))

    r2 = lambda v: v.reshape(1, -1)

    h1, as1, ad1 = _tc1(x, fc1_W, r2(fc1_b), gat1_W, ps1, pd1, N)
    as1p = jnp.pad(as1, ((0, NPAD - N), (0, 0)))
    ad1p = jnp.pad(ad1, ((0, NPAD - N), (0, 0)))

    raw1a, den1 = _gat_agg(h1[:, :192], as1p, ad1p, src2, dst2, z192, z16,
                           48, 0, rng)
    raw1b, _ = _gat_agg(h1[:, 192:], as1p, ad1p, src2, dst2, z192, z16,
                        48, 4, rng)
    raw1 = jnp.concatenate([raw1a, raw1b], axis=1)

    h2, hh2, as2, ad2 = _tc_mid(raw1[:N], den1[:N], x, rep1, r2(gat1_bias),
                                r2(bn1_g), r2(bn1_b), fc5_W, r2(fc5_b),
                                gat2_W, ps2, pd2, N, last=False)
    as2p = jnp.pad(as2, ((0, NPAD - N), (0, 0)))
    ad2p = jnp.pad(ad2, ((0, NPAD - N), (0, 0)))

    raw2, den2 = _gat_agg(hh2, as2p, ad2p, src2, dst2, z256, z16, 256, 0, rng)

    (hf,) = _tc_mid(raw2[:N], den2[:N], h2, rep2, r2(gat2_bias), r2(bn2_g),
                    r2(bn2_b), fc2_W, r2(fc2_b), fc2_W, ps2, pd2, N, last=True)

    x1, x2 = _sc_pairs(hf, pairs, tidp, T3)

    w4 = jnp.pad(fc4_W, ((0, 0), (0, 128 - fc4_W.shape[1])))
    b4 = jnp.pad(fc4_b, (0, 128 - fc4_b.shape[0])).reshape(1, 128)
    out = _tc4((x1[:T], x2[:T]), w4, b4, T)
    return out[:, :fc4_W.shape[1]]


# speedup vs baseline: 11.7205x; 1.5225x over previous
import functools

import jax
import jax.numpy as jnp
import numpy as np
from jax import lax
from jax.experimental import pallas as pl
from jax.experimental.pallas import tpu as pltpu
from jax.experimental.pallas import tpu_sc as plsc

_F32 = jnp.float32
_I32 = jnp.int32

_B = 128
_NW = 32
_RB = 2000


def _tc1(x, w1, b1, gw, ps, pd, N):
    nb = N // _RB

    def body(x_ref, w1_ref, b1_ref, gw_ref, ps_ref, pd_ref,
             h_ref, as_ref, ad_ref):
        xg = jnp.dot(x_ref[...], w1_ref[...],
                     preferred_element_type=_F32, precision=jax.lax.Precision.HIGHEST) + b1_ref[...]
        h = jnp.dot(xg, gw_ref[...], preferred_element_type=_F32, precision=jax.lax.Precision.HIGHEST)
        h_ref[...] = h
        as_ref[...] = jnp.dot(h, ps_ref[...], preferred_element_type=_F32, precision=jax.lax.Precision.HIGHEST)
        ad_ref[...] = jnp.dot(h, pd_ref[...], preferred_element_type=_F32, precision=jax.lax.Precision.HIGHEST)

    D = x.shape[1]
    full = lambda a: pl.BlockSpec(a.shape, lambda i: (0,) * a.ndim)
    return pl.pallas_call(
        body,
        grid=(nb,),
        in_specs=[pl.BlockSpec((_RB, D), lambda i: (i, 0)),
                  full(w1), full(b1), full(gw), full(ps), full(pd)],
        out_specs=[pl.BlockSpec((_RB, D), lambda i: (i, 0)),
                   pl.BlockSpec((_RB, 16), lambda i: (i, 0)),
                   pl.BlockSpec((_RB, 16), lambda i: (i, 0))],
        out_shape=[jax.ShapeDtypeStruct((N, D), _F32),
                   jax.ShapeDtypeStruct((N, 16), _F32),
                   jax.ShapeDtypeStruct((N, 16), _F32)],
    )(x, w1, b1, gw, ps, pd)


def _tc_mid(raw, den, res, rep, gbias, bng, bnb, wn, bn_, gw2, ps2, pd2, N,
            last):
    nb = N // _RB
    Din = raw.shape[1]
    Dout = wn.shape[1]

    def body(raw_ref, den_ref, res_ref, rep_ref, gb_ref, g_ref, b_ref,
             wn_ref, bn_ref, gw2_ref, ps2_ref, pd2_ref, *o_refs):
        acc_s, acc_ss = o_refs[-2], o_refs[-1]
        outs = o_refs[:-2]
        p = pl.program_id(0)
        i = pl.program_id(1)
        dexp = jnp.dot(den_ref[...], rep_ref[...], preferred_element_type=_F32, precision=jax.lax.Precision.HIGHEST)
        y = raw_ref[...] / (dexp + 1e-16) + gb_ref[...]

        @pl.when(p == 0)
        def _():
            bs = jnp.sum(y, axis=0, keepdims=True)
            bss = jnp.sum(y * y, axis=0, keepdims=True)

            @pl.when(i == 0)
            def _():
                acc_s[...] = bs
                acc_ss[...] = bss

            @pl.when(i > 0)
            def _():
                acc_s[...] += bs
                acc_ss[...] += bss

        @pl.when(p == 1)
        def _():
            mu = acc_s[...] / N
            var = acc_ss[...] / N - mu * mu
            ybn = g_ref[...] * (y - mu) * lax.rsqrt(var + 1e-5) + b_ref[...]
            h = jnp.maximum(res_ref[...] + ybn, 0.0)
            h2 = jnp.dot(h, wn_ref[...], preferred_element_type=_F32, precision=jax.lax.Precision.HIGHEST) + bn_ref[...]
            outs[0][...] = h2
            if not last:
                hh = jnp.dot(h2, gw2_ref[...], preferred_element_type=_F32, precision=jax.lax.Precision.HIGHEST)
                outs[1][...] = hh
                outs[2][...] = jnp.dot(hh, ps2_ref[...],
                                       preferred_element_type=_F32, precision=jax.lax.Precision.HIGHEST)
                outs[3][...] = jnp.dot(hh, pd2_ref[...],
                                       preferred_element_type=_F32, precision=jax.lax.Precision.HIGHEST)

    full = lambda a: pl.BlockSpec(a.shape, lambda p, i: (0,) * a.ndim)
    out_shape = [jax.ShapeDtypeStruct((N, Dout), _F32)]
    out_specs = [pl.BlockSpec((_RB, Dout), lambda p, i: (i, 0))]
    if not last:
        out_shape += [jax.ShapeDtypeStruct((N, Dout), _F32),
                      jax.ShapeDtypeStruct((N, 16), _F32),
                      jax.ShapeDtypeStruct((N, 16), _F32)]
        out_specs += [pl.BlockSpec((_RB, Dout), lambda p, i: (i, 0)),
                      pl.BlockSpec((_RB, 16), lambda p, i: (i, 0)),
                      pl.BlockSpec((_RB, 16), lambda p, i: (i, 0))]
    return pl.pallas_call(
        body,
        grid=(2, nb),
        in_specs=[pl.BlockSpec((_RB, Din), lambda p, i: (i, 0)),
                  pl.BlockSpec((_RB, 16), lambda p, i: (i, 0)),
                  pl.BlockSpec((_RB, Din), lambda p, i: (i, 0)),
                  full(rep), full(gbias), full(bng), full(bnb),
                  full(wn), full(bn_), full(gw2), full(ps2), full(pd2)],
        out_specs=out_specs,
        out_shape=out_shape,
        scratch_shapes=[pltpu.VMEM((1, Din), _F32), pltpu.VMEM((1, Din), _F32)],
    )(raw, den, res, rep, gbias, bng, bnb, wn, bn_, gw2, ps2, pd2)


def _tc4(p, w, b, T):
    nb = T // _RB

    def body(x1_ref, x2_ref, w_ref, b_ref, o_ref):
        o_ref[...] = jnp.dot(x1_ref[...] * x2_ref[...], w_ref[...],
                             preferred_element_type=_F32, precision=jax.lax.Precision.HIGHEST) + b_ref[...]

    x1, x2 = p
    full = lambda a: pl.BlockSpec(a.shape, lambda i: (0,) * a.ndim)
    return pl.pallas_call(
        body,
        grid=(nb,),
        in_specs=[pl.BlockSpec((_RB, 256), lambda i: (i, 0)),
                  pl.BlockSpec((_RB, 256), lambda i: (i, 0)),
                  full(w), full(b)],
        out_specs=pl.BlockSpec((_RB, 128), lambda i: (i, 0)),
        out_shape=jax.ShapeDtypeStruct((T, 128), _F32),
    )(x1, x2, w, b)


def _gat_agg(h, as_p, ad_p, src2d, zD, z16, CH, hd0, rng, Bk):
    D = h.shape[1]
    E2P = src2d.shape[1]
    nch = E2P // (Bk * 16)
    assert nch % 2 == 0
    rps = rng // 16
    slab_rows = rng + 8
    heads = D // CH
    mesh = plsc.VectorSubcoreMesh(core_axis_name="c", subcore_axis_name="s")

    buf = lambda: (pltpu.VMEM((2, Bk), _I32),
                   pltpu.VMEM((Bk,), _I32),
                   pltpu.VMEM((Bk, 16), _F32),
                   pltpu.VMEM((Bk, 16), _F32),
                   pltpu.VMEM((Bk, 16), _F32),
                   pltpu.VMEM((Bk, D), _F32))

    @functools.partial(
        pl.kernel,
        out_type=(jax.ShapeDtypeStruct((2 * rng, D), _F32),
                  jax.ShapeDtypeStruct((2 * rng, 16), _F32)),
        mesh=mesh,
        compiler_params=pltpu.CompilerParams(use_tc_tiling_on_sc=False),
        scratch_types=[
            *buf(), *buf(),
            pltpu.VMEM_SHARED((slab_rows, D), _F32),
            pltpu.VMEM_SHARED((slab_rows, 16), _F32),
            pltpu.SemaphoreType.DMA, pltpu.SemaphoreType.DMA,
            pltpu.SemaphoreType.DMA, pltpu.SemaphoreType.DMA,
            pltpu.SemaphoreType.DMA, pltpu.SemaphoreType.DMA,
        ],
    )
    def k(h_hbm, as_hbm, ad_hbm, sd_hbm, zD_hbm, z16_hbm,
          raw_hbm, den_hbm,
          et0, idxl0, asr0, adr0, wbuf0, hrows0,
          et1, idxl1, asr1, adr1, wbuf1, hrows1,
          slab, dslab, sa0, sb0, sh0, sa1, sb1, sh1):
        c = lax.axis_index("c")
        s = lax.axis_index("s")
        base = c * rng
        row0 = s * rps
        slots = ((et0, idxl0, asr0, adr0, wbuf0, hrows0, sa0, sb0, sh0),
                 (et1, idxl1, asr1, adr1, wbuf1, hrows1, sa1, sb1, sh1))

        def issue(g, slot):
            et, _, asr, adr, _, hrows, sa, sb, sh = slots[slot]
            e0 = (g * 16 + s) * Bk
            pltpu.sync_copy(sd_hbm.at[:, pl.ds(e0, Bk)], et)
            pltpu.async_copy(as_hbm.at[et.at[0]], asr, sa)
            pltpu.async_copy(ad_hbm.at[et.at[1]], adr, sb)
            pltpu.async_copy(h_hbm.at[et.at[0]], hrows, sh)

        def process(slot):
            et, idxl, asr, adr, wbuf, hrows, sa, sb, sh = slots[slot]
            pltpu.make_async_copy(as_hbm.at[et.at[0]], asr, sa).wait()
            pltpu.make_async_copy(ad_hbm.at[et.at[1]], adr, sb).wait()
            pltpu.make_async_copy(h_hbm.at[et.at[0]], hrows, sh).wait()

            @plsc.parallel_loop(0, Bk, step=16)
            def _(i):
                dl = et[1, pl.ds(i, 16)] - base
                inr = (dl >= 0) & (dl < rng)
                idxl[pl.ds(i, 16)] = jnp.where(inr, dl, rng)

            @plsc.parallel_loop(0, Bk, unroll=2)
            def _(e):
                l16 = asr[e, pl.ds(0, 16)] + adr[e, pl.ds(0, 16)]
                w16 = jnp.exp(jnp.maximum(l16, 0.2 * l16))
                wbuf[e, pl.ds(0, 16)] = w16
                for hd in range(heads):
                    wsc = w16[hd0 + hd]
                    for j in range(CH // 16):
                        sl = pl.ds(hd * CH + j * 16, 16)
                        hrows[e, sl] = hrows[e, sl] * wsc

            pltpu.sync_copy(hrows, slab.at[idxl], add=True)
            pltpu.sync_copy(wbuf, dslab.at[idxl], add=True)

        pltpu.sync_copy(zD_hbm, slab.at[pl.ds(row0, rps)])
        pltpu.sync_copy(z16_hbm, dslab.at[pl.ds(row0, rps)])
        plsc.subcore_barrier()

        issue(0, 0)

        @pl.loop(0, nch, step=2)
        def _(g):
            issue(g + 1, 1)
            process(0)

            @pl.when(g + 2 < nch)
            def _():
                issue(g + 2, 0)

            process(1)

        plsc.subcore_barrier()
        pltpu.sync_copy(slab.at[pl.ds(row0, rps)],
                        raw_hbm.at[pl.ds(base + row0, rps)])
        pltpu.sync_copy(dslab.at[pl.ds(row0, rps)],
                        den_hbm.at[pl.ds(base + row0, rps)])

    return k(h, as_p, ad_p, src2d, zD, z16)


def _sc_pairs(hf, pairs, tid, T3):
    nch = T3 // (_B * _NW)
    mesh = plsc.VectorSubcoreMesh(core_axis_name="c", subcore_axis_name="s")

    @functools.partial(
        pl.kernel,
        out_type=(jax.ShapeDtypeStruct((T3, 256), _F32),
                  jax.ShapeDtypeStruct((T3, 256), _F32)),
        mesh=mesh,
        compiler_params=pltpu.CompilerParams(use_tc_tiling_on_sc=False,
                                             needs_layout_passes=False),
        scratch_types=[
            pltpu.VMEM((_B,), _I32),
            pltpu.VMEM((_B, 16), _I32),
            pltpu.VMEM((_B,), _I32),
            pltpu.VMEM((_B,), _I32),
            pltpu.VMEM((_B, 256), _F32),
            pltpu.VMEM((_B, 256), _F32),
        ],
    )
    def k(hf_hbm, pairs_hbm, tid_hbm, o1_hbm, o2_hbm,
          tidv, prow, srcv, dstv, x1, x2):
        c = lax.axis_index("c")
        s = lax.axis_index("s")
        wid = s * 2 + c

        @pl.loop(0, nch)
        def _(g):
            t0 = (g * _NW + wid) * _B
            pltpu.sync_copy(tid_hbm.at[pl.ds(t0, _B)], tidv)
            pltpu.sync_copy(pairs_hbm.at[tidv], prow)

            @pl.loop(0, _B, step=16)
            def _(i):
                ri = jnp.arange(16, dtype=_I32) + i
                srcv[pl.ds(i, 16)] = plsc.load_gather(
                    prow, [ri, jnp.zeros((16,), _I32)])
                dstv[pl.ds(i, 16)] = plsc.load_gather(
                    prow, [ri, jnp.ones((16,), _I32)])

            pltpu.sync_copy(hf_hbm.at[srcv], x1)
            pltpu.sync_copy(hf_hbm.at[dstv], x2)
            pltpu.sync_copy(x1, o1_hbm.at[pl.ds(t0, _B)])
            pltpu.sync_copy(x2, o2_hbm.at[pl.ds(t0, _B)])

    return k(hf, pairs, tid)


def _blockdiag(att, heads, ch):
    eye = jnp.eye(heads, dtype=_F32)
    P = (eye[:, None, :] * att[:, :, None]).reshape(heads * ch, heads)
    return jnp.pad(P, ((0, 0), (0, 16 - heads)))


def kernel(x, edge_index, train_edge_id, fc1_W, fc1_b, gat1_W, gat1_as,
           gat1_ad, gat1_bias, bn1_g, bn1_b, fc5_W, fc5_b, gat2_W, gat2_as,
           gat2_ad, gat2_bias, bn2_g, bn2_b, fc2_W, fc2_b, fc4_W, fc4_b):
    N = x.shape[0]
    E = edge_index.shape[1]
    T = train_edge_id.shape[0]
    rng = -(-(-(-N // 2)) // 128) * 128
    NPAD = 2 * rng
    pad_dst = N + 8
    E2 = E + N
    E2P = -(-E2 // (_B * _NW)) * (_B * _NW)
    T3 = -(-T // (_B * _NW)) * (_B * _NW)

    loop = jnp.arange(N, dtype=_I32)
    src2 = jnp.concatenate([edge_index[0], loop,
                            jnp.zeros((E2P - E2,), _I32)])
    dst2 = jnp.concatenate([edge_index[1], loop,
                            jnp.full((E2P - E2,), pad_dst, _I32)])
    src2d = jnp.stack([src2, dst2])
    pairs = jnp.zeros((E, 16), _I32)
    pairs = pairs.at[:, 0].set(edge_index[0]).at[:, 1].set(edge_index[1])
    tidp = jnp.concatenate([train_edge_id, jnp.zeros((T3 - T,), _I32)])

    rps = rng // 16
    z192 = jnp.zeros((rps, 192), _F32)
    z256 = jnp.zeros((rps, 256), _F32)
    z16 = jnp.zeros((rps, 16), _F32)

    ps1 = _blockdiag(gat1_as, 8, 48)
    pd1 = _blockdiag(gat1_ad, 8, 48)
    ps2 = _blockdiag(gat2_as, 1, 256)
    pd2 = _blockdiag(gat2_ad, 1, 256)
    rep1 = jnp.asarray(np.pad(np.kron(np.eye(8, dtype=np.float32),
                                      np.ones((1, 48), np.float32)),
                              ((0, 8), (0, 0))))
    rep2 = jnp.asarray(np.pad(np.ones((1, 256), np.float32), ((0, 15), (0, 0))))

    r2 = lambda v: v.reshape(1, -1)

    h1, as1, ad1 = _tc1(x, fc1_W, r2(fc1_b), gat1_W, ps1, pd1, N)
    as1p = jnp.pad(as1, ((0, NPAD - N), (0, 0)))
    ad1p = jnp.pad(ad1, ((0, NPAD - N), (0, 0)))

    raw1a, den1 = _gat_agg(h1[:, :192], as1p, ad1p, src2d, z192, z16,
                           48, 0, rng, 128)
    raw1b, _ = _gat_agg(h1[:, 192:], as1p, ad1p, src2d, z192, z16,
                        48, 4, rng, 128)
    raw1 = jnp.concatenate([raw1a, raw1b], axis=1)

    h2, hh2, as2, ad2 = _tc_mid(raw1[:N], den1[:N], x, rep1, r2(gat1_bias),
                                r2(bn1_g), r2(bn1_b), fc5_W, r2(fc5_b),
                                gat2_W, ps2, pd2, N, last=False)
    as2p = jnp.pad(as2, ((0, NPAD - N), (0, 0)))
    ad2p = jnp.pad(ad2, ((0, NPAD - N), (0, 0)))

    raw2, den2 = _gat_agg(hh2, as2p, ad2p, src2d, z256, z16, 256, 0, rng, 64)

    (hf,) = _tc_mid(raw2[:N], den2[:N], h2, rep2, r2(gat2_bias), r2(bn2_g),
                    r2(bn2_b), fc2_W, r2(fc2_b), fc2_W, ps2, pd2, N, last=True)

    x1, x2 = _sc_pairs(hf, pairs, tidp, T3)

    w4 = jnp.pad(fc4_W, ((0, 0), (0, 128 - fc4_W.shape[1])))
    b4 = jnp.pad(fc4_b, (0, 128 - fc4_b.shape[0])).reshape(1, 128)
    out = _tc4((x1[:T], x2[:T]), w4, b4, T)
    return out[:, :fc4_W.shape[1]]


# speedup vs baseline: 11.9071x; 1.0159x over previous
import functools

import jax
import jax.numpy as jnp
import numpy as np
from jax import lax
from jax.experimental import pallas as pl
from jax.experimental.pallas import tpu as pltpu
from jax.experimental.pallas import tpu_sc as plsc

_F32 = jnp.float32
_I32 = jnp.int32

_B = 128
_NW = 32
_RB = 2000


def _tc1(x, w1, b1, gw, ps, pd, N):
    nb = N // _RB

    def body(x_ref, w1_ref, b1_ref, gw_ref, ps_ref, pd_ref,
             h_ref, as_ref, ad_ref):
        xg = jnp.dot(x_ref[...], w1_ref[...],
                     preferred_element_type=_F32, precision=jax.lax.Precision.HIGHEST) + b1_ref[...]
        h = jnp.dot(xg, gw_ref[...], preferred_element_type=_F32, precision=jax.lax.Precision.HIGHEST)
        h_ref[...] = h
        as_ref[...] = jnp.dot(h, ps_ref[...], preferred_element_type=_F32, precision=jax.lax.Precision.HIGHEST)
        ad_ref[...] = jnp.dot(h, pd_ref[...], preferred_element_type=_F32, precision=jax.lax.Precision.HIGHEST)

    D = x.shape[1]
    full = lambda a: pl.BlockSpec(a.shape, lambda i: (0,) * a.ndim)
    return pl.pallas_call(
        body,
        grid=(nb,),
        in_specs=[pl.BlockSpec((_RB, D), lambda i: (i, 0)),
                  full(w1), full(b1), full(gw), full(ps), full(pd)],
        out_specs=[pl.BlockSpec((_RB, D), lambda i: (i, 0)),
                   pl.BlockSpec((_RB, 16), lambda i: (i, 0)),
                   pl.BlockSpec((_RB, 16), lambda i: (i, 0))],
        out_shape=[jax.ShapeDtypeStruct((N, D), _F32),
                   jax.ShapeDtypeStruct((N, 16), _F32),
                   jax.ShapeDtypeStruct((N, 16), _F32)],
    )(x, w1, b1, gw, ps, pd)


def _tc_mid(raw, den, res, rep, gbias, bng, bnb, wn, bn_, gw2, ps2, pd2, N,
            last):
    nb = N // _RB
    Din = raw.shape[1]
    Dout = wn.shape[1]

    def body(raw_ref, den_ref, res_ref, rep_ref, gb_ref, g_ref, b_ref,
             wn_ref, bn_ref, gw2_ref, ps2_ref, pd2_ref, *o_refs):
        acc_s, acc_ss = o_refs[-2], o_refs[-1]
        outs = o_refs[:-2]
        p = pl.program_id(0)
        i = pl.program_id(1)
        dexp = jnp.dot(den_ref[...], rep_ref[...], preferred_element_type=_F32, precision=jax.lax.Precision.HIGHEST)
        y = raw_ref[...] / (dexp + 1e-16) + gb_ref[...]

        @pl.when(p == 0)
        def _():
            bs = jnp.sum(y, axis=0, keepdims=True)
            bss = jnp.sum(y * y, axis=0, keepdims=True)

            @pl.when(i == 0)
            def _():
                acc_s[...] = bs
                acc_ss[...] = bss

            @pl.when(i > 0)
            def _():
                acc_s[...] += bs
                acc_ss[...] += bss

        @pl.when(p == 1)
        def _():
            mu = acc_s[...] / N
            var = acc_ss[...] / N - mu * mu
            ybn = g_ref[...] * (y - mu) * lax.rsqrt(var + 1e-5) + b_ref[...]
            h = jnp.maximum(res_ref[...] + ybn, 0.0)
            h2 = jnp.dot(h, wn_ref[...], preferred_element_type=_F32, precision=jax.lax.Precision.HIGHEST) + bn_ref[...]
            outs[0][...] = h2
            if not last:
                hh = jnp.dot(h2, gw2_ref[...], preferred_element_type=_F32, precision=jax.lax.Precision.HIGHEST)
                outs[1][...] = hh
                outs[2][...] = jnp.dot(hh, ps2_ref[...],
                                       preferred_element_type=_F32, precision=jax.lax.Precision.HIGHEST)
                outs[3][...] = jnp.dot(hh, pd2_ref[...],
                                       preferred_element_type=_F32, precision=jax.lax.Precision.HIGHEST)

    full = lambda a: pl.BlockSpec(a.shape, lambda p, i: (0,) * a.ndim)
    out_shape = [jax.ShapeDtypeStruct((N, Dout), _F32)]
    out_specs = [pl.BlockSpec((_RB, Dout), lambda p, i: (i, 0))]
    if not last:
        out_shape += [jax.ShapeDtypeStruct((N, Dout), _F32),
                      jax.ShapeDtypeStruct((N, 16), _F32),
                      jax.ShapeDtypeStruct((N, 16), _F32)]
        out_specs += [pl.BlockSpec((_RB, Dout), lambda p, i: (i, 0)),
                      pl.BlockSpec((_RB, 16), lambda p, i: (i, 0)),
                      pl.BlockSpec((_RB, 16), lambda p, i: (i, 0))]
    return pl.pallas_call(
        body,
        grid=(2, nb),
        in_specs=[pl.BlockSpec((_RB, Din), lambda p, i: (i, 0)),
                  pl.BlockSpec((_RB, 16), lambda p, i: (i, 0)),
                  pl.BlockSpec((_RB, Din), lambda p, i: (i, 0)),
                  full(rep), full(gbias), full(bng), full(bnb),
                  full(wn), full(bn_), full(gw2), full(ps2), full(pd2)],
        out_specs=out_specs,
        out_shape=out_shape,
        scratch_shapes=[pltpu.VMEM((1, Din), _F32), pltpu.VMEM((1, Din), _F32)],
    )(raw, den, res, rep, gbias, bng, bnb, wn, bn_, gw2, ps2, pd2)


def _tc4(p, w, b, T):
    nb = T // _RB

    def body(x1_ref, x2_ref, w_ref, b_ref, o_ref):
        o_ref[...] = jnp.dot(x1_ref[...] * x2_ref[...], w_ref[...],
                             preferred_element_type=_F32, precision=jax.lax.Precision.HIGHEST) + b_ref[...]

    x1, x2 = p
    full = lambda a: pl.BlockSpec(a.shape, lambda i: (0,) * a.ndim)
    return pl.pallas_call(
        body,
        grid=(nb,),
        in_specs=[pl.BlockSpec((_RB, 256), lambda i: (i, 0)),
                  pl.BlockSpec((_RB, 256), lambda i: (i, 0)),
                  full(w), full(b)],
        out_specs=pl.BlockSpec((_RB, 128), lambda i: (i, 0)),
        out_shape=jax.ShapeDtypeStruct((T, 128), _F32),
    )(x1, x2, w, b)


def _gat_agg(h_parts, as_p, ad_p, src2d, zD, z16, CH, hd0s, rng, Bk):
    np_ = len(h_parts)
    D = h_parts[0].shape[1]
    E2P = src2d.shape[1]
    nch = E2P // (Bk * 16)
    assert nch % 2 == 0
    rps = rng // 16
    slab_rows = rng + 8
    heads = D // CH
    mesh = plsc.VectorSubcoreMesh(core_axis_name="c", subcore_axis_name="s")

    buf = lambda: (pltpu.VMEM((2, Bk), _I32),
                   pltpu.VMEM((Bk,), _I32),
                   pltpu.VMEM((Bk, 16), _F32),
                   pltpu.VMEM((Bk, 16), _F32),
                   pltpu.VMEM((Bk, 16), _F32),
                   pltpu.VMEM((Bk, D), _F32))

    @functools.partial(
        pl.kernel,
        out_type=tuple(jax.ShapeDtypeStruct((2 * rng, D), _F32)
                       for _ in range(np_))
                 + (jax.ShapeDtypeStruct((2 * rng, 16), _F32),),
        mesh=mesh,
        compiler_params=pltpu.CompilerParams(use_tc_tiling_on_sc=False),
        scratch_types=[
            *buf(), *buf(),
            pltpu.VMEM_SHARED((slab_rows, D), _F32),
            pltpu.VMEM_SHARED((slab_rows, 16), _F32),
        ] + [pltpu.SemaphoreType.DMA] * 8,
    )
    def k(*refs):
        h_hbms = refs[:np_]
        as_hbm, ad_hbm, sd_hbm, zD_hbm, z16_hbm = refs[np_:np_ + 5]
        raw_hbms = refs[np_ + 5:2 * np_ + 5]
        den_hbm = refs[2 * np_ + 5]
        scr = refs[2 * np_ + 6:]
        slab, dslab = scr[12], scr[13]
        sems = scr[14:]
        slots = (tuple(scr[0:6]) + (sems[0], sems[1], sems[2], sems[6]),
                 tuple(scr[6:12]) + (sems[3], sems[4], sems[5], sems[7]))
        c = lax.axis_index("c")
        s = lax.axis_index("s")
        base = c * rng
        row0 = s * rps

        def issue(g, slot, ph):
            et, _, asr, adr, _, hrows, sa, sb, sh, _ = slots[slot]
            e0 = (g * 16 + s) * Bk
            pltpu.sync_copy(sd_hbm.at[:, pl.ds(e0, Bk)], et)
            pltpu.async_copy(as_hbm.at[et.at[0]], asr, sa)
            pltpu.async_copy(ad_hbm.at[et.at[1]], adr, sb)
            pltpu.async_copy(h_hbms[ph].at[et.at[0]], hrows, sh)

        def process(slot, ph):
            et, idxl, asr, adr, wbuf, hrows, sa, sb, sh, sc_ = slots[slot]
            hd0 = hd0s[ph]
            with_den = ph == 0
            pltpu.make_async_copy(as_hbm.at[et.at[0]], asr, sa).wait()
            pltpu.make_async_copy(ad_hbm.at[et.at[1]], adr, sb).wait()
            pltpu.make_async_copy(h_hbms[ph].at[et.at[0]], hrows, sh).wait()

            @plsc.parallel_loop(0, Bk, step=16)
            def _(i):
                dl = et[1, pl.ds(i, 16)] - base
                inr = (dl >= 0) & (dl < rng)
                idxl[pl.ds(i, 16)] = jnp.where(inr, dl, rng)

            @plsc.parallel_loop(0, Bk, unroll=4)
            def _(e):
                l16 = asr[e, pl.ds(0, 16)] + adr[e, pl.ds(0, 16)]
                w16 = jnp.exp(jnp.maximum(l16, 0.2 * l16))
                if with_den:
                    wbuf[e, pl.ds(0, 16)] = w16
                for hd in range(heads):
                    wsc = w16[hd0 + hd]
                    for j in range(CH // 16):
                        sl = pl.ds(hd * CH + j * 16, 16)
                        hrows[e, sl] = hrows[e, sl] * wsc

            pltpu.async_copy(hrows, slab.at[idxl], sc_, add=True)
            if with_den:
                pltpu.async_copy(wbuf, dslab.at[idxl], sc_, add=True)
            pltpu.make_async_copy(hrows, slab.at[idxl], sc_).wait()
            if with_den:
                pltpu.make_async_copy(wbuf, dslab.at[idxl], sc_).wait()

        pltpu.sync_copy(zD_hbm, slab.at[pl.ds(row0, rps)])
        pltpu.sync_copy(z16_hbm, dslab.at[pl.ds(row0, rps)])
        plsc.subcore_barrier()

        for ph in range(np_):
            issue(0, 0, ph)

            @pl.loop(0, nch, step=2)
            def _(g):
                issue(g + 1, 1, ph)
                process(0, ph)

                @pl.when(g + 2 < nch)
                def _():
                    issue(g + 2, 0, ph)

                process(1, ph)

            plsc.subcore_barrier()
            pltpu.sync_copy(slab.at[pl.ds(row0, rps)],
                            raw_hbms[ph].at[pl.ds(base + row0, rps)])
            if ph == 0:
                pltpu.sync_copy(dslab.at[pl.ds(row0, rps)],
                                den_hbm.at[pl.ds(base + row0, rps)])
            if ph + 1 < np_:
                pltpu.sync_copy(zD_hbm, slab.at[pl.ds(row0, rps)])
                plsc.subcore_barrier()

    out = k(*h_parts, as_p, ad_p, src2d, zD, z16)
    return out[:np_], out[np_]


def _sc_pairs(hf, pairs, tid, T3):
    nch = T3 // (_B * _NW)
    mesh = plsc.VectorSubcoreMesh(core_axis_name="c", subcore_axis_name="s")

    @functools.partial(
        pl.kernel,
        out_type=(jax.ShapeDtypeStruct((T3, 256), _F32),
                  jax.ShapeDtypeStruct((T3, 256), _F32)),
        mesh=mesh,
        compiler_params=pltpu.CompilerParams(use_tc_tiling_on_sc=False,
                                             needs_layout_passes=False),
        scratch_types=[
            pltpu.VMEM((_B,), _I32),
            pltpu.VMEM((_B, 16), _I32),
            pltpu.VMEM((_B,), _I32),
            pltpu.VMEM((_B,), _I32),
            pltpu.VMEM((_B, 256), _F32),
            pltpu.VMEM((_B, 256), _F32),
        ],
    )
    def k(hf_hbm, pairs_hbm, tid_hbm, o1_hbm, o2_hbm,
          tidv, prow, srcv, dstv, x1, x2):
        c = lax.axis_index("c")
        s = lax.axis_index("s")
        wid = s * 2 + c

        @pl.loop(0, nch)
        def _(g):
            t0 = (g * _NW + wid) * _B
            pltpu.sync_copy(tid_hbm.at[pl.ds(t0, _B)], tidv)
            pltpu.sync_copy(pairs_hbm.at[tidv], prow)

            @pl.loop(0, _B, step=16)
            def _(i):
                ri = jnp.arange(16, dtype=_I32) + i
                srcv[pl.ds(i, 16)] = plsc.load_gather(
                    prow, [ri, jnp.zeros((16,), _I32)])
                dstv[pl.ds(i, 16)] = plsc.load_gather(
                    prow, [ri, jnp.ones((16,), _I32)])

            pltpu.sync_copy(hf_hbm.at[srcv], x1)
            pltpu.sync_copy(hf_hbm.at[dstv], x2)
            pltpu.sync_copy(x1, o1_hbm.at[pl.ds(t0, _B)])
            pltpu.sync_copy(x2, o2_hbm.at[pl.ds(t0, _B)])

    return k(hf, pairs, tid)


def _blockdiag(att, heads, ch):
    eye = jnp.eye(heads, dtype=_F32)
    P = (eye[:, None, :] * att[:, :, None]).reshape(heads * ch, heads)
    return jnp.pad(P, ((0, 0), (0, 16 - heads)))


def kernel(x, edge_index, train_edge_id, fc1_W, fc1_b, gat1_W, gat1_as,
           gat1_ad, gat1_bias, bn1_g, bn1_b, fc5_W, fc5_b, gat2_W, gat2_as,
           gat2_ad, gat2_bias, bn2_g, bn2_b, fc2_W, fc2_b, fc4_W, fc4_b):
    N = x.shape[0]
    E = edge_index.shape[1]
    T = train_edge_id.shape[0]
    rng = -(-(-(-N // 2)) // 128) * 128
    NPAD = 2 * rng
    pad_dst = N + 8
    E2 = E + N
    E2P = -(-E2 // (_B * _NW)) * (_B * _NW)
    T3 = -(-T // (_B * _NW)) * (_B * _NW)

    loop = jnp.arange(N, dtype=_I32)
    src2 = jnp.concatenate([edge_index[0], loop,
                            jnp.zeros((E2P - E2,), _I32)])
    dst2 = jnp.concatenate([edge_index[1], loop,
                            jnp.full((E2P - E2,), pad_dst, _I32)])
    src2d = jnp.stack([src2, dst2])
    pairs = jnp.zeros((E, 16), _I32)
    pairs = pairs.at[:, 0].set(edge_index[0]).at[:, 1].set(edge_index[1])
    tidp = jnp.concatenate([train_edge_id, jnp.zeros((T3 - T,), _I32)])

    rps = rng // 16
    z192 = jnp.zeros((rps, 192), _F32)
    z256 = jnp.zeros((rps, 256), _F32)
    z16 = jnp.zeros((rps, 16), _F32)

    ps1 = _blockdiag(gat1_as, 8, 48)
    pd1 = _blockdiag(gat1_ad, 8, 48)
    ps2 = _blockdiag(gat2_as, 1, 256)
    pd2 = _blockdiag(gat2_ad, 1, 256)
    rep1 = jnp.asarray(np.pad(np.kron(np.eye(8, dtype=np.float32),
                                      np.ones((1, 48), np.float32)),
                              ((0, 8), (0, 0))))
    rep2 = jnp.asarray(np.pad(np.ones((1, 256), np.float32), ((0, 15), (0, 0))))

    r2 = lambda v: v.reshape(1, -1)

    h1, as1, ad1 = _tc1(x, fc1_W, r2(fc1_b), gat1_W, ps1, pd1, N)
    as1p = jnp.pad(as1, ((0, NPAD - N), (0, 0)))
    ad1p = jnp.pad(ad1, ((0, NPAD - N), (0, 0)))

    (raw1a, raw1b), den1 = _gat_agg((h1[:, :192], h1[:, 192:]), as1p, ad1p,
                                    src2d, z192, z16, 48, (0, 4), rng, 128)
    raw1 = jnp.concatenate([raw1a, raw1b], axis=1)

    h2, hh2, as2, ad2 = _tc_mid(raw1[:N], den1[:N], x, rep1, r2(gat1_bias),
                                r2(bn1_g), r2(bn1_b), fc5_W, r2(fc5_b),
                                gat2_W, ps2, pd2, N, last=False)
    as2p = jnp.pad(as2, ((0, NPAD - N), (0, 0)))
    ad2p = jnp.pad(ad2, ((0, NPAD - N), (0, 0)))

    (raw2,), den2 = _gat_agg((hh2,), as2p, ad2p, src2d, z256, z16,
                             256, (0,), rng, 64)

    (hf,) = _tc_mid(raw2[:N], den2[:N], h2, rep2, r2(gat2_bias), r2(bn2_g),
                    r2(bn2_b), fc2_W, r2(fc2_b), fc2_W, ps2, pd2, N, last=True)

    x1, x2 = _sc_pairs(hf, pairs, tidp, T3)

    w4 = jnp.pad(fc4_W, ((0, 0), (0, 128 - fc4_W.shape[1])))
    b4 = jnp.pad(fc4_b, (0, 128 - fc4_b.shape[0])).reshape(1, 128)
    out = _tc4((x1[:T], x2[:T]), w4, b4, T)
    return out[:, :fc4_W.shape[1]]


# speedup vs baseline: 13.4159x; 1.1267x over previous
import functools

import jax
import jax.numpy as jnp
import numpy as np
from jax import lax
from jax.experimental import pallas as pl
from jax.experimental.pallas import tpu as pltpu
from jax.experimental.pallas import tpu_sc as plsc

_F32 = jnp.float32
_I32 = jnp.int32

_B = 128
_NW = 32
_RB = 2000


def _tc1(x, w1, b1, gw, ps, pd, N):
    nb = N // _RB

    def body(x_ref, w1_ref, b1_ref, gw_ref, ps_ref, pd_ref,
             h_ref, as_ref, ad_ref):
        xg = jnp.dot(x_ref[...], w1_ref[...],
                     preferred_element_type=_F32, precision=jax.lax.Precision.HIGHEST) + b1_ref[...]
        h = jnp.dot(xg, gw_ref[...], preferred_element_type=_F32, precision=jax.lax.Precision.HIGHEST)
        h_ref[...] = h
        as_ref[...] = jnp.dot(h, ps_ref[...], preferred_element_type=_F32, precision=jax.lax.Precision.HIGHEST)
        ad_ref[...] = jnp.dot(h, pd_ref[...], preferred_element_type=_F32, precision=jax.lax.Precision.HIGHEST)

    D = x.shape[1]
    full = lambda a: pl.BlockSpec(a.shape, lambda i: (0,) * a.ndim)
    return pl.pallas_call(
        body,
        grid=(nb,),
        in_specs=[pl.BlockSpec((_RB, D), lambda i: (i, 0)),
                  full(w1), full(b1), full(gw), full(ps), full(pd)],
        out_specs=[pl.BlockSpec((_RB, D), lambda i: (i, 0)),
                   pl.BlockSpec((_RB, 16), lambda i: (i, 0)),
                   pl.BlockSpec((_RB, 16), lambda i: (i, 0))],
        out_shape=[jax.ShapeDtypeStruct((N, D), _F32),
                   jax.ShapeDtypeStruct((N, 16), _F32),
                   jax.ShapeDtypeStruct((N, 16), _F32)],
    )(x, w1, b1, gw, ps, pd)


def _tc_mid(raw, den, res, rep, gbias, bng, bnb, wn, bn_, gw2, ps2, pd2, N,
            last):
    nb = N // _RB
    Din = raw.shape[1]
    Dout = wn.shape[1]

    def body(raw_ref, den_ref, res_ref, rep_ref, gb_ref, g_ref, b_ref,
             wn_ref, bn_ref, gw2_ref, ps2_ref, pd2_ref, *o_refs):
        acc_s, acc_ss = o_refs[-2], o_refs[-1]
        outs = o_refs[:-2]
        p = pl.program_id(0)
        i = pl.program_id(1)
        dexp = jnp.dot(den_ref[...], rep_ref[...], preferred_element_type=_F32, precision=jax.lax.Precision.HIGHEST)
        y = raw_ref[...] / (dexp + 1e-16) + gb_ref[...]

        @pl.when(p == 0)
        def _():
            bs = jnp.sum(y, axis=0, keepdims=True)
            bss = jnp.sum(y * y, axis=0, keepdims=True)

            @pl.when(i == 0)
            def _():
                acc_s[...] = bs
                acc_ss[...] = bss

            @pl.when(i > 0)
            def _():
                acc_s[...] += bs
                acc_ss[...] += bss

        @pl.when(p == 1)
        def _():
            mu = acc_s[...] / N
            var = acc_ss[...] / N - mu * mu
            ybn = g_ref[...] * (y - mu) * lax.rsqrt(var + 1e-5) + b_ref[...]
            h = jnp.maximum(res_ref[...] + ybn, 0.0)
            h2 = jnp.dot(h, wn_ref[...], preferred_element_type=_F32, precision=jax.lax.Precision.HIGHEST) + bn_ref[...]
            outs[0][...] = h2
            if not last:
                hh = jnp.dot(h2, gw2_ref[...], preferred_element_type=_F32, precision=jax.lax.Precision.HIGHEST)
                outs[1][...] = hh
                outs[2][...] = jnp.dot(hh, ps2_ref[...],
                                       preferred_element_type=_F32, precision=jax.lax.Precision.HIGHEST)
                outs[3][...] = jnp.dot(hh, pd2_ref[...],
                                       preferred_element_type=_F32, precision=jax.lax.Precision.HIGHEST)

    full = lambda a: pl.BlockSpec(a.shape, lambda p, i: (0,) * a.ndim)
    out_shape = [jax.ShapeDtypeStruct((N, Dout), _F32)]
    out_specs = [pl.BlockSpec((_RB, Dout), lambda p, i: (i, 0))]
    if not last:
        out_shape += [jax.ShapeDtypeStruct((N, Dout), _F32),
                      jax.ShapeDtypeStruct((N, 16), _F32),
                      jax.ShapeDtypeStruct((N, 16), _F32)]
        out_specs += [pl.BlockSpec((_RB, Dout), lambda p, i: (i, 0)),
                      pl.BlockSpec((_RB, 16), lambda p, i: (i, 0)),
                      pl.BlockSpec((_RB, 16), lambda p, i: (i, 0))]
    return pl.pallas_call(
        body,
        grid=(2, nb),
        in_specs=[pl.BlockSpec((_RB, Din), lambda p, i: (i, 0)),
                  pl.BlockSpec((_RB, 16), lambda p, i: (i, 0)),
                  pl.BlockSpec((_RB, Din), lambda p, i: (i, 0)),
                  full(rep), full(gbias), full(bng), full(bnb),
                  full(wn), full(bn_), full(gw2), full(ps2), full(pd2)],
        out_specs=out_specs,
        out_shape=out_shape,
        scratch_shapes=[pltpu.VMEM((1, Din), _F32), pltpu.VMEM((1, Din), _F32)],
    )(raw, den, res, rep, gbias, bng, bnb, wn, bn_, gw2, ps2, pd2)


def _tc4(p, w, b, T):
    nb = T // _RB

    def body(x1_ref, x2_ref, w_ref, b_ref, o_ref):
        o_ref[...] = jnp.dot(x1_ref[...] * x2_ref[...], w_ref[...],
                             preferred_element_type=_F32, precision=jax.lax.Precision.HIGHEST) + b_ref[...]

    x1, x2 = p
    full = lambda a: pl.BlockSpec(a.shape, lambda i: (0,) * a.ndim)
    return pl.pallas_call(
        body,
        grid=(nb,),
        in_specs=[pl.BlockSpec((_RB, 256), lambda i: (i, 0)),
                  pl.BlockSpec((_RB, 256), lambda i: (i, 0)),
                  full(w), full(b)],
        out_specs=pl.BlockSpec((_RB, 128), lambda i: (i, 0)),
        out_shape=jax.ShapeDtypeStruct((T, 128), _F32),
    )(x1, x2, w, b)


def _sc_partition(src2d, rng, cap, pad_dst):
    E2P = src2d.shape[1]
    per_s = E2P // 16
    PB = 512
    nchp = per_s // PB
    mesh = plsc.VectorSubcoreMesh(core_axis_name="c", subcore_axis_name="s")

    @functools.partial(
        pl.kernel,
        out_type=(jax.ShapeDtypeStruct((2, 16, 2, cap), _I32),
                  jax.ShapeDtypeStruct((2, 16, 16), _I32)),
        mesh=mesh,
        compiler_params=pltpu.CompilerParams(use_tc_tiling_on_sc=False,
                                             needs_layout_passes=False),
        scratch_types=[
            pltpu.VMEM((PB,), _I32),
            pltpu.VMEM((PB,), _I32),
            pltpu.VMEM((cap,), _I32),
            pltpu.VMEM((cap,), _I32),
            pltpu.VMEM((16,), _I32),
        ],
    )
    def k(sd_hbm, pd_hbm, cnt_hbm, es, ed, stg_s, stg_d, ctv):
        c = lax.axis_index("c")
        s = lax.axis_index("s")
        base = c * rng
        lane = jnp.arange(16, dtype=_I32)

        @pl.loop(0, nchp, init_carry=jnp.int32(0))
        def off_fin(g, off):
            e0 = s * per_s + g * PB
            pltpu.sync_copy(sd_hbm.at[0, pl.ds(e0, PB)], es)
            pltpu.sync_copy(sd_hbm.at[1, pl.ds(e0, PB)], ed)

            @pl.loop(0, PB, step=16, init_carry=off)
            def off2(i, o):
                src16 = es[pl.ds(i, 16)]
                dst16 = ed[pl.ds(i, 16)]
                dl = dst16 - base
                m = (dl >= 0) & (dl < rng)
                plsc.store_compressed(stg_s.at[pl.ds(o, 16)], src16, mask=m)
                plsc.store_compressed(stg_d.at[pl.ds(o, 16)], dst16, mask=m)
                pc = plsc.all_reduce_population_count(m)
                if pc.ndim:
                    pc = pc[0]
                return o + pc

            return off2

        @pl.loop(0, 256, step=16)
        def _(kk):
            stg_s[pl.ds(off_fin + kk, 16)] = jnp.zeros((16,), _I32)
            stg_d[pl.ds(off_fin + kk, 16)] = jnp.full((16,), pad_dst, _I32)

        padded = ((off_fin + 255) // 256) * 256
        ctv[pl.ds(0, 16)] = jnp.where(lane == 0, padded, 0)
        pltpu.sync_copy(stg_s, pd_hbm.at[c, s, 0])
        pltpu.sync_copy(stg_d, pd_hbm.at[c, s, 1])
        pltpu.sync_copy(ctv, cnt_hbm.at[c, s])

    return k(src2d)


def _gat_agg(h_parts, as_p, ad_p, pd, cnts, zD, z16, CH, hd0s, rng, Bk, cap):
    np_ = len(h_parts)
    D = h_parts[0].shape[1]
    rps = rng // 16
    slab_rows = rng + 8
    heads = D // CH
    mesh = plsc.VectorSubcoreMesh(core_axis_name="c", subcore_axis_name="s")

    buf = lambda: (pltpu.VMEM((2, Bk), _I32),
                   pltpu.VMEM((Bk,), _I32),
                   pltpu.VMEM((Bk, 16), _F32),
                   pltpu.VMEM((Bk, 16), _F32),
                   pltpu.VMEM((Bk, 16), _F32),
                   pltpu.VMEM((Bk, D), _F32))

    @functools.partial(
        pl.kernel,
        out_type=tuple(jax.ShapeDtypeStruct((2 * rng, D), _F32)
                       for _ in range(np_))
                 + (jax.ShapeDtypeStruct((2 * rng, 16), _F32),),
        mesh=mesh,
        compiler_params=pltpu.CompilerParams(use_tc_tiling_on_sc=False),
        scratch_types=[
            *buf(), *buf(),
            pltpu.VMEM((16,), _I32),
            pltpu.VMEM_SHARED((slab_rows, D), _F32),
            pltpu.VMEM_SHARED((slab_rows, 16), _F32),
        ] + [pltpu.SemaphoreType.DMA] * 8,
    )
    def k(*refs):
        h_hbms = refs[:np_]
        as_hbm, ad_hbm, pd_hbm, cnt_hbm, zD_hbm, z16_hbm = refs[np_:np_ + 6]
        raw_hbms = refs[np_ + 6:2 * np_ + 6]
        den_hbm = refs[2 * np_ + 6]
        scr = refs[2 * np_ + 7:]
        ctv = scr[12]
        slab, dslab = scr[13], scr[14]
        sems = scr[15:]
        slots = (tuple(scr[0:6]) + (sems[0], sems[1], sems[2], sems[6]),
                 tuple(scr[6:12]) + (sems[3], sems[4], sems[5], sems[7]))
        c = lax.axis_index("c")
        s = lax.axis_index("s")
        base = c * rng
        row0 = s * rps

        def issue(g, slot, ph):
            et, _, asr, adr, _, hrows, sa, sb, sh, _ = slots[slot]
            e0 = g * Bk
            pltpu.sync_copy(pd_hbm.at[c, s, :, pl.ds(e0, Bk)], et)
            pltpu.async_copy(as_hbm.at[et.at[0]], asr, sa)
            pltpu.async_copy(ad_hbm.at[et.at[1]], adr, sb)
            pltpu.async_copy(h_hbms[ph].at[et.at[0]], hrows, sh)

        def process(slot, ph):
            et, idxl, asr, adr, wbuf, hrows, sa, sb, sh, sc_ = slots[slot]
            hd0 = hd0s[ph]
            with_den = ph == 0
            pltpu.make_async_copy(as_hbm.at[et.at[0]], asr, sa).wait()
            pltpu.make_async_copy(ad_hbm.at[et.at[1]], adr, sb).wait()
            pltpu.make_async_copy(h_hbms[ph].at[et.at[0]], hrows, sh).wait()

            @plsc.parallel_loop(0, Bk, step=16)
            def _(i):
                dl = et[1, pl.ds(i, 16)] - base
                inr = (dl >= 0) & (dl < rng)
                idxl[pl.ds(i, 16)] = jnp.where(inr, dl, rng)

            @plsc.parallel_loop(0, Bk, unroll=4)
            def _(e):
                l16 = asr[e, pl.ds(0, 16)] + adr[e, pl.ds(0, 16)]
                w16 = jnp.exp(jnp.maximum(l16, 0.2 * l16))
                if with_den:
                    wbuf[e, pl.ds(0, 16)] = w16
                for hd in range(heads):
                    wsc = w16[hd0 + hd]
                    for j in range(CH // 16):
                        sl = pl.ds(hd * CH + j * 16, 16)
                        hrows[e, sl] = hrows[e, sl] * wsc

            pltpu.async_copy(hrows, slab.at[idxl], sc_, add=True)
            if with_den:
                pltpu.async_copy(wbuf, dslab.at[idxl], sc_, add=True)
            pltpu.make_async_copy(hrows, slab.at[idxl], sc_).wait()
            if with_den:
                pltpu.make_async_copy(wbuf, dslab.at[idxl], sc_).wait()

        pltpu.sync_copy(zD_hbm, slab.at[pl.ds(row0, rps)])
        pltpu.sync_copy(z16_hbm, dslab.at[pl.ds(row0, rps)])
        pltpu.sync_copy(cnt_hbm.at[c, s], ctv)
        cnt = ctv[pl.ds(0, 16)][0]
        nch = cnt // Bk
        plsc.subcore_barrier()

        for ph in range(np_):
            issue(0, 0, ph)

            @pl.loop(0, nch, step=2)
            def _(g):
                issue(g + 1, 1, ph)
                process(0, ph)

                @pl.when(g + 2 < nch)
                def _():
                    issue(g + 2, 0, ph)

                process(1, ph)

            plsc.subcore_barrier()
            pltpu.sync_copy(slab.at[pl.ds(row0, rps)],
                            raw_hbms[ph].at[pl.ds(base + row0, rps)])
            if ph == 0:
                pltpu.sync_copy(dslab.at[pl.ds(row0, rps)],
                                den_hbm.at[pl.ds(base + row0, rps)])
            if ph + 1 < np_:
                pltpu.sync_copy(zD_hbm, slab.at[pl.ds(row0, rps)])
                plsc.subcore_barrier()

    out = k(*h_parts, as_p, ad_p, pd, cnts, zD, z16)
    return out[:np_], out[np_]


def _sc_pairs(hf, pairs, tid, T3):
    nch = T3 // (_B * _NW)
    mesh = plsc.VectorSubcoreMesh(core_axis_name="c", subcore_axis_name="s")

    @functools.partial(
        pl.kernel,
        out_type=(jax.ShapeDtypeStruct((T3, 256), _F32),
                  jax.ShapeDtypeStruct((T3, 256), _F32)),
        mesh=mesh,
        compiler_params=pltpu.CompilerParams(use_tc_tiling_on_sc=False,
                                             needs_layout_passes=False),
        scratch_types=[
            pltpu.VMEM((_B,), _I32),
            pltpu.VMEM((_B, 16), _I32),
            pltpu.VMEM((_B,), _I32),
            pltpu.VMEM((_B,), _I32),
            pltpu.VMEM((_B, 256), _F32),
            pltpu.VMEM((_B, 256), _F32),
        ],
    )
    def k(hf_hbm, pairs_hbm, tid_hbm, o1_hbm, o2_hbm,
          tidv, prow, srcv, dstv, x1, x2):
        c = lax.axis_index("c")
        s = lax.axis_index("s")
        wid = s * 2 + c

        @pl.loop(0, nch)
        def _(g):
            t0 = (g * _NW + wid) * _B
            pltpu.sync_copy(tid_hbm.at[pl.ds(t0, _B)], tidv)
            pltpu.sync_copy(pairs_hbm.at[tidv], prow)

            @pl.loop(0, _B, step=16)
            def _(i):
                ri = jnp.arange(16, dtype=_I32) + i
                srcv[pl.ds(i, 16)] = plsc.load_gather(
                    prow, [ri, jnp.zeros((16,), _I32)])
                dstv[pl.ds(i, 16)] = plsc.load_gather(
                    prow, [ri, jnp.ones((16,), _I32)])

            pltpu.sync_copy(hf_hbm.at[srcv], x1)
            pltpu.sync_copy(hf_hbm.at[dstv], x2)
            pltpu.sync_copy(x1, o1_hbm.at[pl.ds(t0, _B)])
            pltpu.sync_copy(x2, o2_hbm.at[pl.ds(t0, _B)])

    return k(hf, pairs, tid)


def _blockdiag(att, heads, ch):
    eye = jnp.eye(heads, dtype=_F32)
    P = (eye[:, None, :] * att[:, :, None]).reshape(heads * ch, heads)
    return jnp.pad(P, ((0, 0), (0, 16 - heads)))


def kernel(x, edge_index, train_edge_id, fc1_W, fc1_b, gat1_W, gat1_as,
           gat1_ad, gat1_bias, bn1_g, bn1_b, fc5_W, fc5_b, gat2_W, gat2_as,
           gat2_ad, gat2_bias, bn2_g, bn2_b, fc2_W, fc2_b, fc4_W, fc4_b):
    N = x.shape[0]
    E = edge_index.shape[1]
    T = train_edge_id.shape[0]
    rng = -(-(-(-N // 2)) // 128) * 128
    NPAD = 2 * rng
    pad_dst = N + 8
    E2 = E + N
    E2P = -(-E2 // (_B * _NW)) * (_B * _NW)
    T3 = -(-T // (_B * _NW)) * (_B * _NW)

    loop = jnp.arange(N, dtype=_I32)
    src2 = jnp.concatenate([edge_index[0], loop,
                            jnp.zeros((E2P - E2,), _I32)])
    dst2 = jnp.concatenate([edge_index[1], loop,
                            jnp.full((E2P - E2,), pad_dst, _I32)])
    src2d = jnp.stack([src2, dst2])
    cap = E2P // 16 + 256
    pd, cnts = _sc_partition(src2d, rng, cap, pad_dst)
    pairs = jnp.zeros((E, 16), _I32)
    pairs = pairs.at[:, 0].set(edge_index[0]).at[:, 1].set(edge_index[1])
    tidp = jnp.concatenate([train_edge_id, jnp.zeros((T3 - T,), _I32)])

    rps = rng // 16
    z192 = jnp.zeros((rps, 192), _F32)
    z256 = jnp.zeros((rps, 256), _F32)
    z16 = jnp.zeros((rps, 16), _F32)

    ps1 = _blockdiag(gat1_as, 8, 48)
    pd1 = _blockdiag(gat1_ad, 8, 48)
    ps2 = _blockdiag(gat2_as, 1, 256)
    pd2 = _blockdiag(gat2_ad, 1, 256)
    rep1 = jnp.asarray(np.pad(np.kron(np.eye(8, dtype=np.float32),
                                      np.ones((1, 48), np.float32)),
                              ((0, 8), (0, 0))))
    rep2 = jnp.asarray(np.pad(np.ones((1, 256), np.float32), ((0, 15), (0, 0))))

    r2 = lambda v: v.reshape(1, -1)

    h1, as1, ad1 = _tc1(x, fc1_W, r2(fc1_b), gat1_W, ps1, pd1, N)
    as1p = jnp.pad(as1, ((0, NPAD - N), (0, 0)))
    ad1p = jnp.pad(ad1, ((0, NPAD - N), (0, 0)))

    (raw1a, raw1b), den1 = _gat_agg((h1[:, :192], h1[:, 192:]), as1p, ad1p,
                                    pd, cnts, z192, z16, 48, (0, 4), rng, 128,
                                    cap)
    raw1 = jnp.concatenate([raw1a, raw1b], axis=1)

    h2, hh2, as2, ad2 = _tc_mid(raw1[:N], den1[:N], x, rep1, r2(gat1_bias),
                                r2(bn1_g), r2(bn1_b), fc5_W, r2(fc5_b),
                                gat2_W, ps2, pd2, N, last=False)
    as2p = jnp.pad(as2, ((0, NPAD - N), (0, 0)))
    ad2p = jnp.pad(ad2, ((0, NPAD - N), (0, 0)))

    (raw2,), den2 = _gat_agg((hh2,), as2p, ad2p, pd, cnts, z256, z16,
                             256, (0,), rng, 64, cap)

    (hf,) = _tc_mid(raw2[:N], den2[:N], h2, rep2, r2(gat2_bias), r2(bn2_g),
                    r2(bn2_b), fc2_W, r2(fc2_b), fc2_W, ps2, pd2, N, last=True)

    x1, x2 = _sc_pairs(hf, pairs, tidp, T3)

    w4 = jnp.pad(fc4_W, ((0, 0), (0, 128 - fc4_W.shape[1])))
    b4 = jnp.pad(fc4_b, (0, 128 - fc4_b.shape[0])).reshape(1, 128)
    out = _tc4((x1[:T], x2[:T]), w4, b4, T)
    return out[:, :fc4_W.shape[1]]


# speedup vs baseline: 13.4181x; 1.0002x over previous
import functools

import jax
import jax.numpy as jnp
import numpy as np
from jax import lax
from jax.experimental import pallas as pl
from jax.experimental.pallas import tpu as pltpu
from jax.experimental.pallas import tpu_sc as plsc

_F32 = jnp.float32
_I32 = jnp.int32

_B = 128
_NW = 32
_RB = 2000


def _tc1(x, w1, b1, gw, ps, pd, N):
    nb = N // _RB

    def body(x_ref, w1_ref, b1_ref, gw_ref, ps_ref, pd_ref,
             ha_ref, hb_ref, as_ref, ad_ref):
        xg = jnp.dot(x_ref[...], w1_ref[...],
                     preferred_element_type=_F32, precision=jax.lax.Precision.HIGHEST) + b1_ref[...]
        h = jnp.dot(xg, gw_ref[...], preferred_element_type=_F32, precision=jax.lax.Precision.HIGHEST)
        ha_ref[...] = h[:, :192]
        hb_ref[...] = h[:, 192:]
        as_ref[...] = jnp.dot(h, ps_ref[...], preferred_element_type=_F32, precision=jax.lax.Precision.HIGHEST)
        ad_ref[...] = jnp.dot(h, pd_ref[...], preferred_element_type=_F32, precision=jax.lax.Precision.HIGHEST)

    D = x.shape[1]
    full = lambda a: pl.BlockSpec(a.shape, lambda i: (0,) * a.ndim)
    return pl.pallas_call(
        body,
        grid=(nb,),
        in_specs=[pl.BlockSpec((_RB, D), lambda i: (i, 0)),
                  full(w1), full(b1), full(gw), full(ps), full(pd)],
        out_specs=[pl.BlockSpec((_RB, 192), lambda i: (i, 0)),
                   pl.BlockSpec((_RB, 192), lambda i: (i, 0)),
                   pl.BlockSpec((_RB, 16), lambda i: (i, 0)),
                   pl.BlockSpec((_RB, 16), lambda i: (i, 0))],
        out_shape=[jax.ShapeDtypeStruct((N, 192), _F32),
                   jax.ShapeDtypeStruct((N, 192), _F32),
                   jax.ShapeDtypeStruct((N, 16), _F32),
                   jax.ShapeDtypeStruct((N, 16), _F32)],
    )(x, w1, b1, gw, ps, pd)


def _tc_mid(raw_parts, den, res, rep, gbias, bng, bnb, wn, bn_, gw2, ps2,
            pd2, N, last):
    nb = N // _RB
    npart = len(raw_parts)
    Din = sum(r.shape[1] for r in raw_parts)
    Dout = wn.shape[1]

    def body(*refs):
        raw_refs = refs[:npart]
        (den_ref, res_ref, rep_ref, gb_ref, g_ref, b_ref,
         wn_ref, bn_ref, gw2_ref, ps2_ref, pd2_ref) = refs[npart:npart + 11]
        o_refs = refs[npart + 11:]
        acc_s, acc_ss = o_refs[-2], o_refs[-1]
        outs = o_refs[:-2]
        p = pl.program_id(0)
        i = pl.program_id(1)
        dexp = jnp.dot(den_ref[...], rep_ref[...], preferred_element_type=_F32, precision=jax.lax.Precision.HIGHEST)
        rawcat = jnp.concatenate([r[...] for r in raw_refs], axis=1)
        y = rawcat / (dexp + 1e-16) + gb_ref[...]

        @pl.when(p == 0)
        def _():
            bs = jnp.sum(y, axis=0, keepdims=True)
            bss = jnp.sum(y * y, axis=0, keepdims=True)

            @pl.when(i == 0)
            def _():
                acc_s[...] = bs
                acc_ss[...] = bss

            @pl.when(i > 0)
            def _():
                acc_s[...] += bs
                acc_ss[...] += bss

        @pl.when(p == 1)
        def _():
            mu = acc_s[...] / N
            var = acc_ss[...] / N - mu * mu
            ybn = g_ref[...] * (y - mu) * lax.rsqrt(var + 1e-5) + b_ref[...]
            h = jnp.maximum(res_ref[...] + ybn, 0.0)
            h2 = jnp.dot(h, wn_ref[...], preferred_element_type=_F32, precision=jax.lax.Precision.HIGHEST) + bn_ref[...]
            outs[0][...] = h2
            if not last:
                hh = jnp.dot(h2, gw2_ref[...], preferred_element_type=_F32, precision=jax.lax.Precision.HIGHEST)
                outs[1][...] = hh
                outs[2][...] = jnp.dot(hh, ps2_ref[...],
                                       preferred_element_type=_F32, precision=jax.lax.Precision.HIGHEST)
                outs[3][...] = jnp.dot(hh, pd2_ref[...],
                                       preferred_element_type=_F32, precision=jax.lax.Precision.HIGHEST)

    full = lambda a: pl.BlockSpec(a.shape, lambda p, i: (0,) * a.ndim)
    out_shape = [jax.ShapeDtypeStruct((N, Dout), _F32)]
    out_specs = [pl.BlockSpec((_RB, Dout), lambda p, i: (i, 0))]
    if not last:
        out_shape += [jax.ShapeDtypeStruct((N, Dout), _F32),
                      jax.ShapeDtypeStruct((N, 16), _F32),
                      jax.ShapeDtypeStruct((N, 16), _F32)]
        out_specs += [pl.BlockSpec((_RB, Dout), lambda p, i: (i, 0)),
                      pl.BlockSpec((_RB, 16), lambda p, i: (i, 0)),
                      pl.BlockSpec((_RB, 16), lambda p, i: (i, 0))]
    return pl.pallas_call(
        body,
        grid=(2, nb),
        in_specs=[pl.BlockSpec((_RB, r.shape[1]), lambda p, i: (i, 0))
                  for r in raw_parts]
                 + [pl.BlockSpec((_RB, 16), lambda p, i: (i, 0)),
                    pl.BlockSpec((_RB, Din), lambda p, i: (i, 0)),
                    full(rep), full(gbias), full(bng), full(bnb),
                    full(wn), full(bn_), full(gw2), full(ps2), full(pd2)],
        out_specs=out_specs,
        out_shape=out_shape,
        scratch_shapes=[pltpu.VMEM((1, Din), _F32), pltpu.VMEM((1, Din), _F32)],
    )(*raw_parts, den, res, rep, gbias, bng, bnb, wn, bn_, gw2, ps2, pd2)


def _tc4(p, w, b, T):
    RB4 = 2048
    nb = T // RB4

    def body(x1_ref, x2_ref, w_ref, b_ref, o_ref):
        o_ref[...] = jnp.dot(x1_ref[...] * x2_ref[...], w_ref[...],
                             preferred_element_type=_F32, precision=jax.lax.Precision.HIGHEST) + b_ref[...]

    x1, x2 = p
    full = lambda a: pl.BlockSpec(a.shape, lambda i: (0,) * a.ndim)
    return pl.pallas_call(
        body,
        grid=(nb,),
        in_specs=[pl.BlockSpec((RB4, 256), lambda i: (i, 0)),
                  pl.BlockSpec((RB4, 256), lambda i: (i, 0)),
                  full(w), full(b)],
        out_specs=pl.BlockSpec((RB4, 128), lambda i: (i, 0)),
        out_shape=jax.ShapeDtypeStruct((T, 128), _F32),
    )(x1, x2, w, b)


def _sc_partition(src2d, rng, cap, pad_dst):
    E2P = src2d.shape[1]
    per_s = E2P // 16
    PB = 512
    nchp = per_s // PB
    mesh = plsc.VectorSubcoreMesh(core_axis_name="c", subcore_axis_name="s")

    @functools.partial(
        pl.kernel,
        out_type=(jax.ShapeDtypeStruct((2, 16, 2, cap), _I32),
                  jax.ShapeDtypeStruct((2, 16, 16), _I32)),
        mesh=mesh,
        compiler_params=pltpu.CompilerParams(use_tc_tiling_on_sc=False,
                                             needs_layout_passes=False),
        scratch_types=[
            pltpu.VMEM((PB,), _I32),
            pltpu.VMEM((PB,), _I32),
            pltpu.VMEM((cap,), _I32),
            pltpu.VMEM((cap,), _I32),
            pltpu.VMEM((16,), _I32),
        ],
    )
    def k(sd_hbm, pd_hbm, cnt_hbm, es, ed, stg_s, stg_d, ctv):
        c = lax.axis_index("c")
        s = lax.axis_index("s")
        base = c * rng
        lane = jnp.arange(16, dtype=_I32)

        @pl.loop(0, nchp, init_carry=jnp.int32(0))
        def off_fin(g, off):
            e0 = s * per_s + g * PB
            pltpu.sync_copy(sd_hbm.at[0, pl.ds(e0, PB)], es)
            pltpu.sync_copy(sd_hbm.at[1, pl.ds(e0, PB)], ed)

            @pl.loop(0, PB, step=16, init_carry=off)
            def off2(i, o):
                src16 = es[pl.ds(i, 16)]
                dst16 = ed[pl.ds(i, 16)]
                dl = dst16 - base
                m = (dl >= 0) & (dl < rng)
                plsc.store_compressed(stg_s.at[pl.ds(o, 16)], src16, mask=m)
                plsc.store_compressed(stg_d.at[pl.ds(o, 16)], dst16, mask=m)
                pc = plsc.all_reduce_population_count(m)
                if pc.ndim:
                    pc = pc[0]
                return o + pc

            return off2

        @pl.loop(0, 256, step=16)
        def _(kk):
            stg_s[pl.ds(off_fin + kk, 16)] = jnp.zeros((16,), _I32)
            stg_d[pl.ds(off_fin + kk, 16)] = jnp.full((16,), pad_dst, _I32)

        padded = ((off_fin + 255) // 256) * 256
        ctv[pl.ds(0, 16)] = jnp.where(lane == 0, padded, 0)
        pltpu.sync_copy(stg_s, pd_hbm.at[c, s, 0])
        pltpu.sync_copy(stg_d, pd_hbm.at[c, s, 1])
        pltpu.sync_copy(ctv, cnt_hbm.at[c, s])

    return k(src2d)


def _gat_agg(h_parts, as_p, ad_p, pd, cnts, zD, z16, CH, hd0s, rng, Bk, cap):
    np_ = len(h_parts)
    D = h_parts[0].shape[1]
    rps = rng // 16
    slab_rows = rng + 8
    heads = D // CH
    mesh = plsc.VectorSubcoreMesh(core_axis_name="c", subcore_axis_name="s")

    buf = lambda: (pltpu.VMEM((2, Bk), _I32),
                   pltpu.VMEM((Bk,), _I32),
                   pltpu.VMEM((Bk, 16), _F32),
                   pltpu.VMEM((Bk, 16), _F32),
                   pltpu.VMEM((Bk, 16), _F32),
                   pltpu.VMEM((Bk, D), _F32))

    @functools.partial(
        pl.kernel,
        out_type=tuple(jax.ShapeDtypeStruct((2 * rng, D), _F32)
                       for _ in range(np_))
                 + (jax.ShapeDtypeStruct((2 * rng, 16), _F32),),
        mesh=mesh,
        compiler_params=pltpu.CompilerParams(use_tc_tiling_on_sc=False),
        scratch_types=[
            *buf(), *buf(),
            pltpu.VMEM((16,), _I32),
            pltpu.VMEM_SHARED((slab_rows, D), _F32),
            pltpu.VMEM_SHARED((slab_rows, 16), _F32),
        ] + [pltpu.SemaphoreType.DMA] * 8,
    )
    def k(*refs):
        h_hbms = refs[:np_]
        as_hbm, ad_hbm, pd_hbm, cnt_hbm, zD_hbm, z16_hbm = refs[np_:np_ + 6]
        raw_hbms = refs[np_ + 6:2 * np_ + 6]
        den_hbm = refs[2 * np_ + 6]
        scr = refs[2 * np_ + 7:]
        ctv = scr[12]
        slab, dslab = scr[13], scr[14]
        sems = scr[15:]
        slots = (tuple(scr[0:6]) + (sems[0], sems[1], sems[2], sems[6]),
                 tuple(scr[6:12]) + (sems[3], sems[4], sems[5], sems[7]))
        c = lax.axis_index("c")
        s = lax.axis_index("s")
        base = c * rng
        row0 = s * rps

        def issue(g, slot, ph):
            et, _, asr, adr, _, hrows, sa, sb, sh, _ = slots[slot]
            e0 = g * Bk
            pltpu.sync_copy(pd_hbm.at[c, s, :, pl.ds(e0, Bk)], et)
            pltpu.async_copy(as_hbm.at[et.at[0]], asr, sa)
            pltpu.async_copy(ad_hbm.at[et.at[1]], adr, sb)
            pltpu.async_copy(h_hbms[ph].at[et.at[0]], hrows, sh)

        def process(slot, ph):
            et, idxl, asr, adr, wbuf, hrows, sa, sb, sh, sc_ = slots[slot]
            hd0 = hd0s[ph]
            with_den = ph == 0
            pltpu.make_async_copy(as_hbm.at[et.at[0]], asr, sa).wait()
            pltpu.make_async_copy(ad_hbm.at[et.at[1]], adr, sb).wait()
            pltpu.make_async_copy(h_hbms[ph].at[et.at[0]], hrows, sh).wait()

            @plsc.parallel_loop(0, Bk, step=16)
            def _(i):
                dl = et[1, pl.ds(i, 16)] - base
                inr = (dl >= 0) & (dl < rng)
                idxl[pl.ds(i, 16)] = jnp.where(inr, dl, rng)

            @plsc.parallel_loop(0, Bk, unroll=4)
            def _(e):
                l16 = asr[e, pl.ds(0, 16)] + adr[e, pl.ds(0, 16)]
                w16 = jnp.exp(jnp.maximum(l16, 0.2 * l16))
                if with_den:
                    wbuf[e, pl.ds(0, 16)] = w16
                for hd in range(heads):
                    wsc = w16[hd0 + hd]
                    for j in range(CH // 16):
                        sl = pl.ds(hd * CH + j * 16, 16)
                        hrows[e, sl] = hrows[e, sl] * wsc

            pltpu.async_copy(hrows, slab.at[idxl], sc_, add=True)
            if with_den:
                pltpu.async_copy(wbuf, dslab.at[idxl], sc_, add=True)
            pltpu.make_async_copy(hrows, slab.at[idxl], sc_).wait()
            if with_den:
                pltpu.make_async_copy(wbuf, dslab.at[idxl], sc_).wait()

        pltpu.sync_copy(zD_hbm, slab.at[pl.ds(row0, rps)])
        pltpu.sync_copy(z16_hbm, dslab.at[pl.ds(row0, rps)])
        pltpu.sync_copy(cnt_hbm.at[c, s], ctv)
        cnt = ctv[pl.ds(0, 16)][0]
        nch = cnt // Bk
        plsc.subcore_barrier()

        for ph in range(np_):
            issue(0, 0, ph)

            @pl.loop(0, nch, step=2)
            def _(g):
                issue(g + 1, 1, ph)
                process(0, ph)

                @pl.when(g + 2 < nch)
                def _():
                    issue(g + 2, 0, ph)

                process(1, ph)

            plsc.subcore_barrier()
            pltpu.sync_copy(slab.at[pl.ds(row0, rps)],
                            raw_hbms[ph].at[pl.ds(base + row0, rps)])
            if ph == 0:
                pltpu.sync_copy(dslab.at[pl.ds(row0, rps)],
                                den_hbm.at[pl.ds(base + row0, rps)])
            if ph + 1 < np_:
                pltpu.sync_copy(zD_hbm, slab.at[pl.ds(row0, rps)])
                plsc.subcore_barrier()

    out = k(*h_parts, as_p, ad_p, pd, cnts, zD, z16)
    return out[:np_], out[np_]


def _sc_pairs(hf, pairs, tid, T3):
    nch = T3 // (_B * _NW)
    mesh = plsc.VectorSubcoreMesh(core_axis_name="c", subcore_axis_name="s")

    @functools.partial(
        pl.kernel,
        out_type=(jax.ShapeDtypeStruct((T3, 256), _F32),
                  jax.ShapeDtypeStruct((T3, 256), _F32)),
        mesh=mesh,
        compiler_params=pltpu.CompilerParams(use_tc_tiling_on_sc=False,
                                             needs_layout_passes=False),
        scratch_types=[
            pltpu.VMEM((_B,), _I32),
            pltpu.VMEM((_B, 16), _I32),
            pltpu.VMEM((_B,), _I32),
            pltpu.VMEM((_B,), _I32),
            pltpu.VMEM((_B, 256), _F32),
            pltpu.VMEM((_B, 256), _F32),
        ],
    )
    def k(hf_hbm, pairs_hbm, tid_hbm, o1_hbm, o2_hbm,
          tidv, prow, srcv, dstv, x1, x2):
        c = lax.axis_index("c")
        s = lax.axis_index("s")
        wid = s * 2 + c

        @pl.loop(0, nch)
        def _(g):
            t0 = (g * _NW + wid) * _B
            pltpu.sync_copy(tid_hbm.at[pl.ds(t0, _B)], tidv)
            pltpu.sync_copy(pairs_hbm.at[tidv], prow)

            @pl.loop(0, _B, step=16)
            def _(i):
                ri = jnp.arange(16, dtype=_I32) + i
                srcv[pl.ds(i, 16)] = plsc.load_gather(
                    prow, [ri, jnp.zeros((16,), _I32)])
                dstv[pl.ds(i, 16)] = plsc.load_gather(
                    prow, [ri, jnp.ones((16,), _I32)])

            pltpu.sync_copy(hf_hbm.at[srcv], x1)
            pltpu.sync_copy(hf_hbm.at[dstv], x2)
            pltpu.sync_copy(x1, o1_hbm.at[pl.ds(t0, _B)])
            pltpu.sync_copy(x2, o2_hbm.at[pl.ds(t0, _B)])

    return k(hf, pairs, tid)


def _blockdiag(att, heads, ch):
    eye = jnp.eye(heads, dtype=_F32)
    P = (eye[:, None, :] * att[:, :, None]).reshape(heads * ch, heads)
    return jnp.pad(P, ((0, 0), (0, 16 - heads)))


def kernel(x, edge_index, train_edge_id, fc1_W, fc1_b, gat1_W, gat1_as,
           gat1_ad, gat1_bias, bn1_g, bn1_b, fc5_W, fc5_b, gat2_W, gat2_as,
           gat2_ad, gat2_bias, bn2_g, bn2_b, fc2_W, fc2_b, fc4_W, fc4_b):
    N = x.shape[0]
    E = edge_index.shape[1]
    T = train_edge_id.shape[0]
    rng = -(-(-(-N // 2)) // 128) * 128
    NPAD = 2 * rng
    pad_dst = N + 8
    E2 = E + N
    E2P = -(-E2 // (_B * _NW)) * (_B * _NW)
    T3 = -(-T // (_B * _NW)) * (_B * _NW)

    loop = jnp.arange(N, dtype=_I32)
    src2 = jnp.concatenate([edge_index[0], loop,
                            jnp.zeros((E2P - E2,), _I32)])
    dst2 = jnp.concatenate([edge_index[1], loop,
                            jnp.full((E2P - E2,), pad_dst, _I32)])
    src2d = jnp.stack([src2, dst2])
    cap = E2P // 16 + 256
    pd, cnts = _sc_partition(src2d, rng, cap, pad_dst)
    pairs = jnp.zeros((E, 16), _I32)
    pairs = pairs.at[:, 0].set(edge_index[0]).at[:, 1].set(edge_index[1])
    tidp = jnp.concatenate([train_edge_id, jnp.zeros((T3 - T,), _I32)])

    rps = rng // 16
    z192 = jnp.zeros((rps, 192), _F32)
    z256 = jnp.zeros((rps, 256), _F32)
    z16 = jnp.zeros((rps, 16), _F32)

    ps1 = _blockdiag(gat1_as, 8, 48)
    pd1 = _blockdiag(gat1_ad, 8, 48)
    ps2 = _blockdiag(gat2_as, 1, 256)
    pd2 = _blockdiag(gat2_ad, 1, 256)
    rep1 = jnp.asarray(np.pad(np.kron(np.eye(8, dtype=np.float32),
                                      np.ones((1, 48), np.float32)),
                              ((0, 8), (0, 0))))
    rep2 = jnp.asarray(np.pad(np.ones((1, 256), np.float32), ((0, 15), (0, 0))))

    r2 = lambda v: v.reshape(1, -1)

    h1a, h1b, as1, ad1 = _tc1(x, fc1_W, r2(fc1_b), gat1_W, ps1, pd1, N)
    as1p = jnp.pad(as1, ((0, NPAD - N), (0, 0)))
    ad1p = jnp.pad(ad1, ((0, NPAD - N), (0, 0)))

    (raw1a, raw1b), den1 = _gat_agg((h1a, h1b), as1p, ad1p,
                                    pd, cnts, z192, z16, 48, (0, 4), rng, 128,
                                    cap)

    h2, hh2, as2, ad2 = _tc_mid((raw1a, raw1b), den1, x, rep1, r2(gat1_bias),
                                r2(bn1_g), r2(bn1_b), fc5_W, r2(fc5_b),
                                gat2_W, ps2, pd2, N, last=False)
    as2p = jnp.pad(as2, ((0, NPAD - N), (0, 0)))
    ad2p = jnp.pad(ad2, ((0, NPAD - N), (0, 0)))

    (raw2,), den2 = _gat_agg((hh2,), as2p, ad2p, pd, cnts, z256, z16,
                             256, (0,), rng, 64, cap)

    (hf,) = _tc_mid((raw2,), den2, h2, rep2, r2(gat2_bias), r2(bn2_g),
                    r2(bn2_b), fc2_W, r2(fc2_b), fc2_W, ps2, pd2, N, last=True)

    x1, x2 = _sc_pairs(hf, pairs, tidp, T3)

    w4 = jnp.pad(fc4_W, ((0, 0), (0, 128 - fc4_W.shape[1])))
    b4 = jnp.pad(fc4_b, (0, 128 - fc4_b.shape[0])).reshape(1, 128)
    out = _tc4((x1, x2), w4, b4, T3)
    return out[:T, :fc4_W.shape[1]]
